# baseline (speedup 1.0000x reference)
import functools

import jax
import jax.numpy as jnp
from jax import lax
from jax.experimental import pallas as pl
from jax.experimental.pallas import tpu as pltpu, tpu_sc as plsc


def kernel(graph, node_embed):
    del graph
    n, d = node_embed.shape
    info = plsc.get_sparse_core_info()
    nc, ns = info.num_cores, info.num_subcores
    nw = nc * ns
    rows_per_w = (n // (8 * nw)) * 8
    tail = n - nw * rows_per_w
    mesh = plsc.VectorSubcoreMesh(core_axis_name="c", subcore_axis_name="s")

    @functools.partial(
        pl.kernel,
        mesh=mesh,
        out_type=jax.ShapeDtypeStruct((n, d), node_embed.dtype),
        scratch_types=[pltpu.SemaphoreType.DMA, pltpu.SemaphoreType.DMA],
    )
    def sc_copy(x_hbm, o_hbm, sem, tail_sem):
        wid = lax.axis_index("s") * nc + lax.axis_index("c")
        base = wid * rows_per_w
        pltpu.async_copy(
            x_hbm.at[pl.ds(base, rows_per_w)],
            o_hbm.at[pl.ds(base, rows_per_w)],
            sem,
        ).start()
        if tail:
            @pl.when(wid == 0)
            def _():
                pltpu.async_copy(
                    x_hbm.at[pl.ds(nw * rows_per_w, tail)],
                    o_hbm.at[pl.ds(nw * rows_per_w, tail)],
                    tail_sem,
                ).wait()
        pltpu.async_copy(
            x_hbm.at[pl.ds(base, rows_per_w)],
            o_hbm.at[pl.ds(base, rows_per_w)],
            sem,
        ).wait()

    return sc_copy(node_embed)


# speedup vs baseline: 2.9428x; 2.9428x over previous
import functools

import jax
import jax.numpy as jnp
from jax import lax
from jax.experimental import pallas as pl
from jax.experimental.pallas import tpu as pltpu, tpu_sc as plsc


def kernel(graph, node_embed):
    del graph
    n, d = node_embed.shape
    info = plsc.get_sparse_core_info()
    nc, ns = info.num_cores, info.num_subcores
    nw = nc * ns
    rows_per_w = (n // (8 * nw)) * 8
    tail = n - nw * rows_per_w
    mesh = plsc.VectorSubcoreMesh(core_axis_name="c", subcore_axis_name="s")

    n_chunks = 5
    ch = rows_per_w // n_chunks
    assert ch % 8 == 0 and ch * n_chunks == rows_per_w

    @functools.partial(
        pl.kernel,
        mesh=mesh,
        out_type=jax.ShapeDtypeStruct((n, d), node_embed.dtype),
        scratch_types=[
            pltpu.SemaphoreType.DMA((n_chunks,)),
            pltpu.SemaphoreType.DMA,
        ],
    )
    def sc_copy(x_hbm, o_hbm, sems, tail_sem):
        wid = lax.axis_index("s") * nc + lax.axis_index("c")
        base = wid * rows_per_w
        descs = []
        for j in range(n_chunks):
            off = base + j * ch
            descs.append(pltpu.async_copy(
                x_hbm.at[pl.ds(off, ch)],
                o_hbm.at[pl.ds(off, ch)],
                sems.at[j],
            ))
        if tail:
            @pl.when(wid == 0)
            def _():
                pltpu.async_copy(
                    x_hbm.at[pl.ds(nw * rows_per_w, tail)],
                    o_hbm.at[pl.ds(nw * rows_per_w, tail)],
                    tail_sem,
                ).wait()
        for desc in descs:
            desc.wait()

    return sc_copy(node_embed)
